# baseline (speedup 1.0000x reference)
import functools

import jax
import jax.numpy as jnp
from jax import lax
from jax.experimental import pallas as pl
from jax.experimental.pallas import tpu as pltpu
from jax.experimental.pallas import tpu_sc as plsc

VOCAB = 1000000
EMB = 32
B = 4096
L = 50
NW = 32
ROWS = 2 * B
ROWS_PER_W = ROWS // NW
CHUNK = 8
N_CHUNKS = ROWS_PER_W // CHUNK
IDX_PER_CHUNK = CHUNK * L


def _accumulate(rows_v, out_v, r):
    for h in (0, 16):
        accs = [jnp.zeros((16,), jnp.float32) for _ in range(4)]
        for j in range(L):
            accs[j % 4] = accs[j % 4] + rows_v[r * L + j, pl.ds(h, 16)]
        out_v[r, pl.ds(h, 16)] = (accs[0] + accs[1]) + (accs[2] + accs[3])


def _make_pool_kernel():
    mesh = plsc.VectorSubcoreMesh(core_axis_name="c", subcore_axis_name="s")

    @functools.partial(
        pl.kernel,
        mesh=mesh,
        out_type=jax.ShapeDtypeStruct((ROWS, EMB), jnp.float32),
        scratch_types=[
            pltpu.VMEM((IDX_PER_CHUNK,), jnp.int32),
            pltpu.VMEM((IDX_PER_CHUNK, EMB), jnp.float32),
            pltpu.VMEM((CHUNK, EMB), jnp.float32),
            pltpu.SemaphoreType.DMA,
        ],
        compiler_params=pltpu.CompilerParams(use_tc_tiling_on_sc=False),
    )
    def pool(table_hbm, idx_hbm, out_hbm, idx_v, rows_v, out_v, sem):
        wid = lax.axis_index("s") * 2 + lax.axis_index("c")
        base_row = wid * ROWS_PER_W

        @pl.loop(0, N_CHUNKS)
        def _(ci):
            row0 = base_row + ci * CHUNK
            pltpu.sync_copy(idx_hbm.at[pl.ds(row0 * L, IDX_PER_CHUNK)], idx_v)
            pltpu.async_copy(table_hbm.at[idx_v], rows_v, sem).wait()
            for r in range(CHUNK):
                _accumulate(rows_v, out_v, r)
            pltpu.sync_copy(out_v, out_hbm.at[pl.ds(row0, CHUNK), :])

    return pool


_pool = _make_pool_kernel()

RP_ROWS = 250
RP_CHUNKS = VOCAB // (4 * RP_ROWS)


def _make_repack_kernel():
    mesh = plsc.VectorSubcoreMesh(core_axis_name="c", subcore_axis_name="s")

    @functools.partial(
        pl.kernel,
        mesh=mesh,
        out_type=jax.ShapeDtypeStruct((VOCAB, EMB), jnp.float32),
        scratch_types=[
            pltpu.VMEM((RP_ROWS, 128), jnp.float32),
            pltpu.VMEM((4 * RP_ROWS, EMB), jnp.float32),
        ],
        compiler_params=pltpu.CompilerParams(use_tc_tiling_on_sc=False),
    )
    def repack(x_hbm, o_hbm, in_v, out_v, ):
        wid = lax.axis_index("s") * 2 + lax.axis_index("c")

        @pl.loop(0, (RP_CHUNKS + NW - 1) // NW)
        def _(k):
            ch = k * NW + wid

            @pl.when(ch < RP_CHUNKS)
            def _():
                pltpu.sync_copy(x_hbm.at[pl.ds(ch * RP_ROWS, RP_ROWS), :], in_v)
                for r in range(RP_ROWS):
                    for q in range(4):
                        for h in (0, 16):
                            out_v[4 * r + q, pl.ds(h, 16)] = (
                                in_v[r, pl.ds(q * EMB + h, 16)])
                pltpu.sync_copy(out_v,
                                o_hbm.at[pl.ds(ch * 4 * RP_ROWS, 4 * RP_ROWS), :])

    return repack


_repack = _make_repack_kernel()


def _mlp_body(p_ref, il_ref, w1a_ref, w1b_ref, b1_ref, w2_ref, b2_ref, o_ref):
    x0 = p_ref[0] / il_ref[0]
    x1 = p_ref[1] / il_ref[1]
    h = jnp.dot(x0, w1a_ref[...], preferred_element_type=jnp.float32)
    h = h + jnp.dot(x1, w1b_ref[...], preferred_element_type=jnp.float32)
    h = jnp.maximum(h + b1_ref[...], 0.0)
    o_ref[...] = jnp.dot(h, w2_ref[...], preferred_element_type=jnp.float32) + b2_ref[...]


def kernel(data, length, embed_table, W1, b1, W2, b2):
    idx_flat = data.reshape(-1)
    table = _repack(embed_table.reshape(-1, 128))
    pooled = _pool(table, idx_flat).reshape(2, B, EMB)
    lenf = length.astype(jnp.float32).reshape(2, B, 1)
    w2p = jnp.pad(W2, ((0, 0), (0, 128 - W2.shape[1])))
    b2p = jnp.pad(b2, (0, 128 - b2.shape[0]))
    out = pl.pallas_call(
        _mlp_body,
        out_shape=jax.ShapeDtypeStruct((B, 128), jnp.float32),
    )(pooled, lenf, W1[:EMB], W1[EMB:], b1.reshape(1, -1),
      w2p, b2p.reshape(1, -1))
    return out[:, :3]


# speedup vs baseline: 1.1306x; 1.1306x over previous
import functools

import jax
import jax.numpy as jnp
from jax import lax
from jax.experimental import pallas as pl
from jax.experimental.pallas import tpu as pltpu
from jax.experimental.pallas import tpu_sc as plsc

VOCAB = 1000000
EMB = 32
B = 4096
L = 50
NW = 32
ROWS = 2 * B
ROWS_PER_W = ROWS // NW
CHUNK = 8
N_CHUNKS = ROWS_PER_W // CHUNK
IDX_PER_CHUNK = CHUNK * L


def _accumulate(rows_v, out_v, r):
    for h in (0, 16):
        accs = [jnp.zeros((16,), jnp.float32) for _ in range(4)]
        for j in range(L):
            accs[j % 4] = accs[j % 4] + rows_v[r * L + j, pl.ds(h, 16)]
        out_v[r, pl.ds(h, 16)] = (accs[0] + accs[1]) + (accs[2] + accs[3])


def _make_pool_kernel():
    mesh = plsc.VectorSubcoreMesh(core_axis_name="c", subcore_axis_name="s")

    @functools.partial(
        pl.kernel,
        mesh=mesh,
        out_type=jax.ShapeDtypeStruct((ROWS, EMB), jnp.float32),
        scratch_types=[
            pltpu.VMEM((IDX_PER_CHUNK,), jnp.int32),
            pltpu.VMEM((IDX_PER_CHUNK, EMB), jnp.float32),
            pltpu.VMEM((CHUNK, EMB), jnp.float32),
            pltpu.SemaphoreType.DMA,
        ],
        compiler_params=pltpu.CompilerParams(use_tc_tiling_on_sc=False),
    )
    def pool(table_hbm, idx_hbm, out_hbm, idx_v, rows_v, out_v, sem):
        wid = lax.axis_index("s") * 2 + lax.axis_index("c")
        base_row = wid * ROWS_PER_W

        @pl.loop(0, N_CHUNKS)
        def _(ci):
            row0 = base_row + ci * CHUNK
            pltpu.sync_copy(idx_hbm.at[pl.ds(row0 * L, IDX_PER_CHUNK)], idx_v)
            pltpu.async_copy(table_hbm.at[idx_v], rows_v, sem).wait()
            for r in range(CHUNK):
                _accumulate(rows_v, out_v, r)
            pltpu.sync_copy(out_v, out_hbm.at[pl.ds(row0, CHUNK), :])

    return pool


_pool = _make_pool_kernel()

DP_BLK = 8000


def _depad_body(x_ref, o_ref):
    x = x_ref[...].reshape(DP_BLK // 4, 4, EMB)
    o_ref[...] = jnp.concatenate([x[:, q, :] for q in range(4)], axis=1)


def _depad(table):
    return pl.pallas_call(
        _depad_body,
        grid=(VOCAB // DP_BLK,),
        in_specs=[pl.BlockSpec((DP_BLK, EMB), lambda i: (i, 0))],
        out_specs=pl.BlockSpec((DP_BLK // 4, 128), lambda i: (i, 0)),
        out_shape=jax.ShapeDtypeStruct((VOCAB // 4, 128), jnp.float32),
    )(table)


def _mlp_body(p_ref, il_ref, w1a_ref, w1b_ref, b1_ref, w2_ref, b2_ref, o_ref):
    x0 = p_ref[0] / il_ref[0]
    x1 = p_ref[1] / il_ref[1]
    h = jnp.dot(x0, w1a_ref[...], preferred_element_type=jnp.float32)
    h = h + jnp.dot(x1, w1b_ref[...], preferred_element_type=jnp.float32)
    h = jnp.maximum(h + b1_ref[...], 0.0)
    o_ref[...] = jnp.dot(h, w2_ref[...], preferred_element_type=jnp.float32) + b2_ref[...]


def kernel(data, length, embed_table, W1, b1, W2, b2):
    idx_flat = data.reshape(-1)
    table = _depad(embed_table).reshape(VOCAB, EMB)
    pooled = _pool(table, idx_flat).reshape(2, B, EMB)
    lenf = length.astype(jnp.float32).reshape(2, B, 1)
    w2p = jnp.pad(W2, ((0, 0), (0, 128 - W2.shape[1])))
    b2p = jnp.pad(b2, (0, 128 - b2.shape[0]))
    out = pl.pallas_call(
        _mlp_body,
        out_shape=jax.ShapeDtypeStruct((B, 128), jnp.float32),
    )(pooled, lenf, W1[:EMB], W1[EMB:], b1.reshape(1, -1),
      w2p, b2p.reshape(1, -1))
    return out[:, :3]


# speedup vs baseline: 1.3348x; 1.1806x over previous
import functools

import jax
import jax.numpy as jnp
from jax import lax
from jax.experimental import pallas as pl
from jax.experimental.pallas import tpu as pltpu
from jax.experimental.pallas import tpu_sc as plsc

VOCAB = 1000000
EMB = 32
B = 4096
L = 50
NW = 32
ROWS = 2 * B
ROWS_PER_W = ROWS // NW
CHUNK = 8
N_CHUNKS = ROWS_PER_W // CHUNK
IDX_PER_CHUNK = CHUNK * L


def _accumulate(rows_v, out_v, r):
    for h in (0, 16):
        accs = [jnp.zeros((16,), jnp.float32) for _ in range(4)]
        for j in range(L):
            accs[j % 4] = accs[j % 4] + rows_v[r * L + j, pl.ds(h, 16)]
        out_v[r, pl.ds(h, 16)] = (accs[0] + accs[1]) + (accs[2] + accs[3])


def _make_pool_kernel():
    mesh = plsc.VectorSubcoreMesh(core_axis_name="c", subcore_axis_name="s")

    @functools.partial(
        pl.kernel,
        mesh=mesh,
        out_type=jax.ShapeDtypeStruct((ROWS, EMB), jnp.float32),
        scratch_types=[
            pltpu.VMEM((IDX_PER_CHUNK,), jnp.int32),
            pltpu.VMEM((IDX_PER_CHUNK,), jnp.int32),
            pltpu.VMEM((IDX_PER_CHUNK, EMB), jnp.float32),
            pltpu.VMEM((IDX_PER_CHUNK, EMB), jnp.float32),
            pltpu.VMEM((CHUNK, EMB), jnp.float32),
            pltpu.SemaphoreType.DMA,
            pltpu.SemaphoreType.DMA,
        ],
        compiler_params=pltpu.CompilerParams(use_tc_tiling_on_sc=False),
    )
    def pool(table3_hbm, idx_hbm, out_hbm,
             idx0_v, idx1_v, rows0_v, rows1_v, out_v, sem0, sem1):
        table_hbm = table3_hbm.at[0]
        wid = lax.axis_index("s") * 2 + lax.axis_index("c")
        base_row = wid * ROWS_PER_W
        bufs = ((idx0_v, rows0_v, sem0), (idx1_v, rows1_v, sem1))

        def fetch(ci, b):
            idx_v, rows_v, sem = bufs[b]
            row0 = base_row + ci * CHUNK
            pltpu.sync_copy(idx_hbm.at[pl.ds(row0 * L, IDX_PER_CHUNK)], idx_v)
            pltpu.async_copy(table_hbm.at[idx_v], rows_v, sem)

        fetch(0, 0)

        @pl.loop(0, N_CHUNKS, step=2)
        def _(ci):
            for b in range(2):
                cur = ci + b

                @pl.when(cur + 1 < N_CHUNKS)
                def _():
                    fetch(cur + 1, 1 - b)

                idx_v, rows_v, sem = bufs[b]
                pltpu.make_async_copy(table_hbm.at[idx_v], rows_v, sem).wait()
                for r in range(CHUNK):
                    _accumulate(rows_v, out_v, r)
                pltpu.sync_copy(
                    out_v, out_hbm.at[pl.ds(base_row + cur * CHUNK, CHUNK), :])

    return pool


_pool = _make_pool_kernel()


def _mlp_body(p_ref, il_ref, w1a_ref, w1b_ref, b1_ref, w2_ref, b2_ref, o_ref):
    x0 = p_ref[0] / il_ref[0]
    x1 = p_ref[1] / il_ref[1]
    h = jnp.dot(x0, w1a_ref[...], preferred_element_type=jnp.float32)
    h = h + jnp.dot(x1, w1b_ref[...], preferred_element_type=jnp.float32)
    h = jnp.maximum(h + b1_ref[...], 0.0)
    o_ref[...] = jnp.dot(h, w2_ref[...], preferred_element_type=jnp.float32) + b2_ref[...]


def kernel(data, length, embed_table, W1, b1, W2, b2):
    idx_flat = data.reshape(-1)
    table3 = embed_table.reshape(1, VOCAB, EMB)
    pooled = _pool(table3, idx_flat).reshape(2, B, EMB)
    lenf = length.astype(jnp.float32).reshape(2, B, 1)
    w2p = jnp.pad(W2, ((0, 0), (0, 128 - W2.shape[1])))
    b2p = jnp.pad(b2, (0, 128 - b2.shape[0]))
    out = pl.pallas_call(
        _mlp_body,
        out_shape=jax.ShapeDtypeStruct((B, 128), jnp.float32),
    )(pooled, lenf, W1[:EMB], W1[EMB:], b1.reshape(1, -1),
      w2p, b2p.reshape(1, -1))
    return out[:, :3]
